# baseline (speedup 1.0000x reference)
import functools

import jax
import jax.numpy as jnp
from jax import lax
from jax.experimental import pallas as pl
from jax.experimental.pallas import tpu as pltpu
from jax.experimental.pallas import tpu_sc as plsc

_NCHUNK = 4
_C = 128


def _make_gather(V, D, B):
  info = plsc.get_sparse_core_info()
  NW = info.num_cores * info.num_subcores
  b_per_w = B // NW
  assert b_per_w == _NCHUNK * _C
  mesh = plsc.VectorSubcoreMesh(core_axis_name="c", subcore_axis_name="s")

  @functools.partial(
      pl.kernel,
      out_type=jax.ShapeDtypeStruct((B, D), jnp.float32),
      mesh=mesh,
      scratch_types=[
          pltpu.VMEM((b_per_w,), jnp.int32),
          pltpu.VMEM((b_per_w, D), jnp.float32),
          pltpu.SemaphoreType.DMA,
          pltpu.SemaphoreType.DMA,
      ],
  )
  def gather_kernel(idx_hbm, table_hbm, out_hbm, idx_v, rows_v,
                    sem_g, sem_s):
    wid = lax.axis_index("s") * info.num_cores + lax.axis_index("c")
    base = wid * b_per_w
    pltpu.sync_copy(idx_hbm.at[pl.ds(base, b_per_w)], idx_v)

    def issue(g, carry):
      v = idx_v[pl.ds(g * 16, 16)]
      for i in range(16):
        r = v[i]
        pltpu.async_copy(
            table_hbm.at[r >> 3, pl.ds(r & 7, 1)],
            rows_v.at[pl.ds(g * 16 + i, 1)], sem_g)
      return carry

    lax.fori_loop(0, b_per_w // 16, issue, 0, unroll=2)

    writes = []
    for ch in range(_NCHUNK):
      pltpu.make_async_copy(
          table_hbm.at[pl.ds(0, _C // 8)],
          rows_v.at[pl.ds(ch * _C, _C)], sem_g).wait()
      writes.append(pltpu.async_copy(
          rows_v.at[pl.ds(ch * _C, _C)],
          out_hbm.at[pl.ds(base + ch * _C, _C)], sem_s))
    for w in writes:
      w.wait()

  return gather_kernel


def kernel(task_ids, table):
  B = task_ids.shape[0]
  V, D = table.shape
  fn = _make_gather(V, D, B)
  return fn(task_ids.astype(jnp.int32), table.reshape(V // 8, 8, D))
